# baseline (speedup 1.0000x reference)
import functools

import jax
import jax.numpy as jnp
from jax import lax
from jax.experimental import pallas as pl
from jax.experimental.pallas import tpu as pltpu
from jax.experimental.pallas import tpu_sc as plsc

_EPS = 1e-06


def _ln_body(x_ref, enc_ref, w_ref, b_ref, o_ref):
    y = x_ref[0] + enc_ref[...]
    mean = jnp.mean(y, axis=-1, keepdims=True)
    yc = y - mean
    var = jnp.mean(yc * yc, axis=-1, keepdims=True)
    o_ref[0] = yc * jax.lax.rsqrt(var + _EPS) * w_ref[...] + b_ref[...]


def _ln_body_split(xa_ref, xb_ref, enca_ref, encb_ref, w_ref, b_ref, o_ref):
    hb = xa_ref.shape[1]
    w = w_ref[...]
    b = b_ref[...]
    for x_ref, enc_ref, lo in ((xa_ref, enca_ref, 0), (xb_ref, encb_ref, hb)):
        y = x_ref[0] + enc_ref[...]
        mean = jnp.mean(y, axis=-1, keepdims=True)
        yc = y - mean
        var = jnp.mean(yc * yc, axis=-1, keepdims=True)
        o_ref[0, pl.ds(lo, hb), :] = (
            yc * jax.lax.rsqrt(var + _EPS) * w + b)


@functools.partial(jax.jit, static_argnames=("bs",))
def _tempo_enc_tc_split(x, enc_table, ln_w, ln_b, bs=2048):
    batch, seq, n_attr = x.shape
    enc = enc_table[:seq]
    w2 = ln_w.reshape(1, n_attr)
    b2 = ln_b.reshape(1, n_attr)
    hb = bs // 2
    grid = (seq // bs, batch)
    return pl.pallas_call(
        _ln_body_split,
        grid=grid,
        in_specs=[
            pl.BlockSpec((1, hb, n_attr), lambda s, b: (b, 2 * s, 0)),
            pl.BlockSpec((1, hb, n_attr), lambda s, b: (b, 2 * s + 1, 0)),
            pl.BlockSpec((hb, n_attr), lambda s, b: (2 * s, 0)),
            pl.BlockSpec((hb, n_attr), lambda s, b: (2 * s + 1, 0)),
            pl.BlockSpec((1, n_attr), lambda s, b: (0, 0)),
            pl.BlockSpec((1, n_attr), lambda s, b: (0, 0)),
        ],
        out_specs=pl.BlockSpec((1, bs, n_attr), lambda s, b: (b, s, 0)),
        out_shape=jax.ShapeDtypeStruct(x.shape, x.dtype),
        compiler_params=pltpu.CompilerParams(
            dimension_semantics=("arbitrary", "arbitrary"),
        ),
    )(x, x, enc, enc, w2, b2)


@functools.partial(jax.jit, static_argnames=("bs",))
def _tempo_enc_tc(x, enc_table, ln_w, ln_b, bs=2048):
    batch, seq, n_attr = x.shape
    enc = enc_table[:seq]
    w2 = ln_w.reshape(1, n_attr)
    b2 = ln_b.reshape(1, n_attr)
    grid = (seq // bs, batch)
    return pl.pallas_call(
        _ln_body,
        grid=grid,
        in_specs=[
            pl.BlockSpec((1, bs, n_attr), lambda s, b: (b, s, 0)),
            pl.BlockSpec((bs, n_attr), lambda s, b: (s, 0)),
            pl.BlockSpec((1, n_attr), lambda s, b: (0, 0)),
            pl.BlockSpec((1, n_attr), lambda s, b: (0, 0)),
        ],
        out_specs=pl.BlockSpec((1, bs, n_attr), lambda s, b: (b, s, 0)),
        out_shape=jax.ShapeDtypeStruct(x.shape, x.dtype),
        compiler_params=pltpu.CompilerParams(
            dimension_semantics=("arbitrary", "arbitrary"),
        ),
    )(x, enc, w2, b2)


def _ln_body2d(x_ref, enc_ref, w_ref, b_ref, o_ref):
    y = x_ref[...] + enc_ref[...]
    mean = jnp.mean(y, axis=-1, keepdims=True)
    yc = y - mean
    var = jnp.mean(yc * yc, axis=-1, keepdims=True)
    o_ref[...] = yc * jax.lax.rsqrt(var + _EPS) * w_ref[...] + b_ref[...]


def _tc_rows(x2, enc, w2, b2, n_tc_rows, seq, n_attr, bs):
    eb = seq // bs
    return pl.pallas_call(
        _ln_body2d,
        grid=(n_tc_rows // bs,),
        in_specs=[
            pl.BlockSpec((bs, n_attr), lambda s: (s, 0)),
            pl.BlockSpec((bs, n_attr), lambda s, _eb=eb: (s % _eb, 0)),
            pl.BlockSpec((1, n_attr), lambda s: (0, 0)),
            pl.BlockSpec((1, n_attr), lambda s: (0, 0)),
        ],
        out_specs=pl.BlockSpec((bs, n_attr), lambda s: (s, 0)),
        out_shape=jax.ShapeDtypeStruct((n_tc_rows, n_attr), jnp.float32),
        compiler_params=pltpu.CompilerParams(
            dimension_semantics=("arbitrary",),
        ),
    )(x2[:n_tc_rows], enc, w2, b2)


_L = 16


def _newton_rsqrt(v):
    i = plsc.bitcast(v, jnp.int32)
    g = plsc.bitcast(jnp.int32(0x5F3759DF) - (i >> 1), jnp.float32)
    for _ in range(3):
        g = g * (1.5 - 0.5 * v * g * g)
    return g


def _sc_body(row_start, rows_per_worker, chunk_rows, n_attr, seq,
             x_hbm, enc_hbm, w_hbm, b_hbm, out_hbm,
             xv0, encv0, xv1, encv1, wv, bv,
             lsem0, lsem1, ssem0, ssem1):
    info = plsc.get_sparse_core_info()
    nc = info.num_cores
    wid = lax.axis_index("s") * nc + lax.axis_index("c")
    out0 = wid * rows_per_worker
    row0 = row_start + out0
    enc0 = lax.rem(row0, seq)
    pltpu.sync_copy(w_hbm, wv)
    pltpu.sync_copy(b_hbm, bv)

    n_chunks_feat = n_attr // _L
    inv_n = 1.0 / n_attr
    n_chunks = rows_per_worker // chunk_rows
    assert n_chunks % 2 == 0 and n_chunks >= 4
    bufs = ((xv0, encv0, lsem0, ssem0), (xv1, encv1, lsem1, ssem1))

    def fetch(c, b):
        xv, encv, lsem, _ = bufs[b]
        base = row0 + c * chunk_rows
        ebase = enc0 + c * chunk_rows
        pltpu.async_copy(x_hbm.at[pl.ds(base, chunk_rows)], xv, lsem)
        pltpu.async_copy(enc_hbm.at[pl.ds(ebase, chunk_rows)], encv, lsem)

    def drain_store(b):
        xv, _, _, ssem = bufs[b]
        pltpu.make_async_copy(
            xv, out_hbm.at[pl.ds(0, chunk_rows)], ssem).wait()

    def compute_store(c, b):
        xv, encv, lsem, ssem = bufs[b]
        pltpu.make_async_copy(x_hbm.at[pl.ds(0, chunk_rows)], xv, lsem).wait()
        pltpu.make_async_copy(
            enc_hbm.at[pl.ds(0, chunk_rows)], encv, lsem).wait()

        def row_body(r, _):
            s = jnp.zeros((_L,), jnp.float32)
            sq = jnp.zeros((_L,), jnp.float32)
            for i in range(n_chunks_feat):
                y = xv[r, pl.ds(i * _L, _L)] + encv[r, pl.ds(i * _L, _L)]
                xv[r, pl.ds(i * _L, _L)] = y
                s = s + y
                sq = sq + y * y
            mean = jnp.sum(s) * inv_n
            var = jnp.sum(sq) * inv_n - mean * mean
            mean_v = jnp.full((_L,), mean, jnp.float32)
            inv_v = _newton_rsqrt(jnp.full((_L,), var + _EPS, jnp.float32))
            scale = inv_v
            for i in range(n_chunks_feat):
                y = xv[r, pl.ds(i * _L, _L)]
                xv[r, pl.ds(i * _L, _L)] = (
                    (y - mean_v) * scale * wv[pl.ds(i * _L, _L)]
                    + bv[pl.ds(i * _L, _L)])
            return 0

        lax.fori_loop(0, chunk_rows, row_body, 0)
        base = out0 + c * chunk_rows
        pltpu.async_copy(xv, out_hbm.at[pl.ds(base, chunk_rows)], ssem)

    fetch(0, 0)
    fetch(1, 1)

    def pair_body(p, _):
        c = p * 2
        for b in range(2):
            compute_store(c + b, b)

            @pl.when(c + b + 2 < n_chunks)
            def _():
                drain_store(b)
                fetch(c + b + 2, b)

            _ = _
        return 0

    lax.fori_loop(0, n_chunks // 2, pair_body, 0)
    drain_store(0)
    drain_store(1)


def _sc_rows(x2, enc, ln_w, ln_b, row_start, n_sc_rows, seq, n_attr,
             chunk_rows=16):
    info = plsc.get_sparse_core_info()
    n_workers = info.num_cores * info.num_subcores
    rows_per_worker = n_sc_rows // n_workers

    mesh = plsc.VectorSubcoreMesh(core_axis_name="c", subcore_axis_name="s")
    body = functools.partial(
        _sc_body, row_start, rows_per_worker, chunk_rows, n_attr, seq)
    return pl.kernel(
        body,
        mesh=mesh,
        out_type=jax.ShapeDtypeStruct((n_sc_rows, n_attr), jnp.float32),
        compiler_params=pltpu.CompilerParams(needs_layout_passes=False),
        scratch_types=[
            pltpu.VMEM((chunk_rows, n_attr), jnp.float32),
            pltpu.VMEM((chunk_rows, n_attr), jnp.float32),
            pltpu.VMEM((chunk_rows, n_attr), jnp.float32),
            pltpu.VMEM((chunk_rows, n_attr), jnp.float32),
            pltpu.VMEM((n_attr,), jnp.float32),
            pltpu.VMEM((n_attr,), jnp.float32),
            pltpu.SemaphoreType.DMA,
            pltpu.SemaphoreType.DMA,
            pltpu.SemaphoreType.DMA,
            pltpu.SemaphoreType.DMA,
        ],
    )(x2, enc, ln_w, ln_b)


_TC_BS = 2048


@jax.jit
def _tempo_enc_2d(x, enc_table, ln_w, ln_b):
    batch, seq, n_attr = x.shape
    n_rows = batch * seq
    x2 = x.reshape(n_rows, n_attr)
    enc = enc_table[:seq]
    w2 = ln_w.reshape(1, n_attr)
    b2 = ln_b.reshape(1, n_attr)
    out = _tc_rows(x2, enc, w2, b2, n_rows, seq, n_attr, _TC_BS)
    return out.reshape(batch, seq, n_attr)


@jax.jit
def _tempo_enc_sc_full(x, enc_table, ln_w, ln_b):
    batch, seq, n_attr = x.shape
    n_rows = batch * seq
    x2 = x.reshape(n_rows, n_attr)
    enc = enc_table[:seq]
    out = _sc_rows(x2, enc, ln_w, ln_b, 0, n_rows, seq, n_attr)
    return out.reshape(batch, seq, n_attr)


def kernel(x, enc_table, ln_w, ln_b):
    return _tempo_enc_sc_full(x, enc_table, ln_w, ln_b)


# speedup vs baseline: 5.0851x; 5.0851x over previous
import functools

import jax
import jax.numpy as jnp
from jax import lax
from jax.experimental import pallas as pl
from jax.experimental.pallas import tpu as pltpu
from jax.experimental.pallas import tpu_sc as plsc

_EPS = 1e-06


def _ln_body(x_ref, enc_ref, w_ref, b_ref, o_ref):
    y = x_ref[0] + enc_ref[...]
    mean = jnp.mean(y, axis=-1, keepdims=True)
    yc = y - mean
    var = jnp.mean(yc * yc, axis=-1, keepdims=True)
    o_ref[0] = yc * jax.lax.rsqrt(var + _EPS) * w_ref[...] + b_ref[...]


@functools.partial(jax.jit, static_argnames=("bs",))
def _tempo_enc_tc(x, enc_table, ln_w, ln_b, bs=2048):
    batch, seq, n_attr = x.shape
    enc = enc_table[:seq]
    w2 = ln_w.reshape(1, n_attr)
    b2 = ln_b.reshape(1, n_attr)
    grid = (seq // bs, batch)
    return pl.pallas_call(
        _ln_body,
        grid=grid,
        in_specs=[
            pl.BlockSpec((1, bs, n_attr), lambda s, b: (b, s, 0)),
            pl.BlockSpec((bs, n_attr), lambda s, b: (s, 0)),
            pl.BlockSpec((1, n_attr), lambda s, b: (0, 0)),
            pl.BlockSpec((1, n_attr), lambda s, b: (0, 0)),
        ],
        out_specs=pl.BlockSpec((1, bs, n_attr), lambda s, b: (b, s, 0)),
        out_shape=jax.ShapeDtypeStruct(x.shape, x.dtype),
        compiler_params=pltpu.CompilerParams(
            dimension_semantics=("arbitrary", "arbitrary"),
        ),
    )(x, enc, w2, b2)


_L = 16


def _newton_rsqrt(v):
    i = plsc.bitcast(v, jnp.int32)
    g = plsc.bitcast(jnp.int32(0x5F3759DF) - (i >> 1), jnp.float32)
    for _ in range(3):
        g = g * (1.5 - 0.5 * v * g * g)
    return g


def _sc_body(row_start, rows_per_worker, chunk_rows, n_attr, seq,
             x_hbm, enc_hbm, w_hbm, b_hbm, out_hbm,
             xv0, encv0, xv1, encv1, wv, bv,
             lsem0, lsem1, ssem0, ssem1):
    info = plsc.get_sparse_core_info()
    nc = info.num_cores
    wid = lax.axis_index("s") * nc + lax.axis_index("c")
    out0 = wid * rows_per_worker
    row0 = row_start + out0
    enc0 = lax.rem(row0, seq)
    pltpu.sync_copy(w_hbm, wv)
    pltpu.sync_copy(b_hbm, bv)

    n_chunks_feat = n_attr // _L
    inv_n = 1.0 / n_attr
    n_chunks = rows_per_worker // chunk_rows
    assert n_chunks % 2 == 0 and n_chunks >= 4
    bufs = ((xv0, encv0, lsem0, ssem0), (xv1, encv1, lsem1, ssem1))

    def fetch(c, b):
        xv, encv, lsem, _ = bufs[b]
        base = row0 + c * chunk_rows
        ebase = enc0 + c * chunk_rows
        pltpu.async_copy(x_hbm.at[pl.ds(base, chunk_rows)], xv, lsem)
        pltpu.async_copy(enc_hbm.at[pl.ds(ebase, chunk_rows)], encv, lsem)

    def drain_store(b):
        xv, _, _, ssem = bufs[b]
        pltpu.make_async_copy(
            xv, out_hbm.at[pl.ds(0, chunk_rows)], ssem).wait()

    def compute_store(c, b):
        xv, encv, lsem, ssem = bufs[b]
        pltpu.make_async_copy(x_hbm.at[pl.ds(0, chunk_rows)], xv, lsem).wait()
        pltpu.make_async_copy(
            enc_hbm.at[pl.ds(0, chunk_rows)], encv, lsem).wait()

        def row_body(r, _):
            s = jnp.zeros((_L,), jnp.float32)
            sq = jnp.zeros((_L,), jnp.float32)
            for i in range(n_chunks_feat):
                y = xv[r, pl.ds(i * _L, _L)] + encv[r, pl.ds(i * _L, _L)]
                xv[r, pl.ds(i * _L, _L)] = y
                s = s + y
                sq = sq + y * y
            mean = jnp.sum(s) * inv_n
            var = jnp.sum(sq) * inv_n - mean * mean
            mean_v = jnp.full((_L,), mean, jnp.float32)
            scale = _newton_rsqrt(jnp.full((_L,), var + _EPS, jnp.float32))
            for i in range(n_chunks_feat):
                y = xv[r, pl.ds(i * _L, _L)]
                xv[r, pl.ds(i * _L, _L)] = (
                    (y - mean_v) * scale * wv[pl.ds(i * _L, _L)]
                    + bv[pl.ds(i * _L, _L)])
            return 0

        lax.fori_loop(0, chunk_rows, row_body, 0)
        base = out0 + c * chunk_rows
        pltpu.async_copy(xv, out_hbm.at[pl.ds(base, chunk_rows)], ssem)

    fetch(0, 0)
    fetch(1, 1)

    def pair_body(p, _):
        c = p * 2
        for b in range(2):
            compute_store(c + b, b)

            @pl.when(c + b + 2 < n_chunks)
            def _():
                drain_store(b)
                fetch(c + b + 2, b)

        return 0

    lax.fori_loop(0, n_chunks // 2, pair_body, 0)
    drain_store(0)
    drain_store(1)


def _sc_rows(x2, enc, ln_w, ln_b, row_start, n_sc_rows, seq, n_attr,
             chunk_rows=16):
    info = plsc.get_sparse_core_info()
    n_workers = info.num_cores * info.num_subcores
    rows_per_worker = n_sc_rows // n_workers

    mesh = plsc.VectorSubcoreMesh(core_axis_name="c", subcore_axis_name="s")
    body = functools.partial(
        _sc_body, row_start, rows_per_worker, chunk_rows, n_attr, seq)
    return pl.kernel(
        body,
        mesh=mesh,
        out_type=jax.ShapeDtypeStruct((n_sc_rows, n_attr), jnp.float32),
        compiler_params=pltpu.CompilerParams(needs_layout_passes=False),
        scratch_types=[
            pltpu.VMEM((chunk_rows, n_attr), jnp.float32),
            pltpu.VMEM((chunk_rows, n_attr), jnp.float32),
            pltpu.VMEM((chunk_rows, n_attr), jnp.float32),
            pltpu.VMEM((chunk_rows, n_attr), jnp.float32),
            pltpu.VMEM((n_attr,), jnp.float32),
            pltpu.VMEM((n_attr,), jnp.float32),
            pltpu.SemaphoreType.DMA,
            pltpu.SemaphoreType.DMA,
            pltpu.SemaphoreType.DMA,
            pltpu.SemaphoreType.DMA,
        ],
    )(x2, enc, ln_w, ln_b)


@jax.jit
def _tempo_enc_sc_full(x, enc_table, ln_w, ln_b):
    batch, seq, n_attr = x.shape
    n_rows = batch * seq
    x2 = x.reshape(n_rows, n_attr)
    enc = enc_table[:seq]
    out = _sc_rows(x2, enc, ln_w, ln_b, 0, n_rows, seq, n_attr)
    return out.reshape(batch, seq, n_attr)


def kernel(x, enc_table, ln_w, ln_b):
    return _tempo_enc_tc(x, enc_table, ln_w, ln_b, bs=2048)


# speedup vs baseline: 5.3387x; 1.0499x over previous
import functools

import jax
import jax.numpy as jnp
from jax import lax
from jax.experimental import pallas as pl
from jax.experimental.pallas import tpu as pltpu
from jax.experimental.pallas import tpu_sc as plsc

_EPS = 1e-06


def _ln_body(x_ref, enc_ref, w_ref, b_ref, o_ref):
    y = x_ref[0] + enc_ref[...]
    mean = jnp.mean(y, axis=-1, keepdims=True)
    yc = y - mean
    var = jnp.mean(yc * yc, axis=-1, keepdims=True)
    o_ref[0] = yc * jax.lax.rsqrt(var + _EPS) * w_ref[...] + b_ref[...]


@functools.partial(jax.jit, static_argnames=("bs",))
def _tempo_enc_tc(x, enc_table, ln_w, ln_b, bs=2048):
    batch, seq, n_attr = x.shape
    enc = enc_table[:seq]
    w2 = ln_w.reshape(1, n_attr)
    b2 = ln_b.reshape(1, n_attr)
    grid = (seq // bs, batch)
    return pl.pallas_call(
        _ln_body,
        grid=grid,
        in_specs=[
            pl.BlockSpec((1, bs, n_attr), lambda s, b: (b, s, 0)),
            pl.BlockSpec((bs, n_attr), lambda s, b: (s, 0)),
            pl.BlockSpec((1, n_attr), lambda s, b: (0, 0)),
            pl.BlockSpec((1, n_attr), lambda s, b: (0, 0)),
        ],
        out_specs=pl.BlockSpec((1, bs, n_attr), lambda s, b: (b, s, 0)),
        out_shape=jax.ShapeDtypeStruct(x.shape, x.dtype),
        compiler_params=pltpu.CompilerParams(
            dimension_semantics=("arbitrary", "arbitrary"),
        ),
    )(x, enc, w2, b2)


_L = 16


def _newton_rsqrt(v):
    i = plsc.bitcast(v, jnp.int32)
    g = plsc.bitcast(jnp.int32(0x5F3759DF) - (i >> 1), jnp.float32)
    for _ in range(3):
        g = g * (1.5 - 0.5 * v * g * g)
    return g


def _sc_body(row_start, rows_per_worker, chunk_rows, n_attr, seq,
             x_hbm, enc_hbm, w_hbm, b_hbm, out_hbm,
             xv0, encv0, xv1, encv1, wv, bv,
             lsem0, lsem1, ssem0, ssem1):
    info = plsc.get_sparse_core_info()
    nc = info.num_cores
    wid = lax.axis_index("s") * nc + lax.axis_index("c")
    out0 = wid * rows_per_worker
    row0 = row_start + out0
    enc0 = lax.rem(row0, seq)
    pltpu.sync_copy(w_hbm, wv)
    pltpu.sync_copy(b_hbm, bv)

    n_chunks_feat = n_attr // _L
    inv_n = 1.0 / n_attr
    n_chunks = rows_per_worker // chunk_rows
    assert n_chunks % 2 == 0 and n_chunks >= 4
    bufs = ((xv0, encv0, lsem0, ssem0), (xv1, encv1, lsem1, ssem1))

    def fetch(c, b):
        xv, encv, lsem, _ = bufs[b]
        base = row0 + c * chunk_rows
        ebase = enc0 + c * chunk_rows
        pltpu.async_copy(x_hbm.at[pl.ds(base, chunk_rows)], xv, lsem)
        pltpu.async_copy(enc_hbm.at[pl.ds(ebase, chunk_rows)], encv, lsem)

    def drain_store(b):
        xv, _, _, ssem = bufs[b]
        pltpu.make_async_copy(
            xv, out_hbm.at[pl.ds(0, chunk_rows)], ssem).wait()

    def compute_store(c, b):
        xv, encv, lsem, ssem = bufs[b]
        pltpu.make_async_copy(x_hbm.at[pl.ds(0, chunk_rows)], xv, lsem).wait()
        pltpu.make_async_copy(
            enc_hbm.at[pl.ds(0, chunk_rows)], encv, lsem).wait()

        def row_body(r, _):
            s = jnp.zeros((_L,), jnp.float32)
            sq = jnp.zeros((_L,), jnp.float32)
            for i in range(n_chunks_feat):
                y = xv[r, pl.ds(i * _L, _L)] + encv[r, pl.ds(i * _L, _L)]
                xv[r, pl.ds(i * _L, _L)] = y
                s = s + y
                sq = sq + y * y
            mean = jnp.sum(s) * inv_n
            var = jnp.sum(sq) * inv_n - mean * mean
            mean_v = jnp.full((_L,), mean, jnp.float32)
            scale = _newton_rsqrt(jnp.full((_L,), var + _EPS, jnp.float32))
            for i in range(n_chunks_feat):
                y = xv[r, pl.ds(i * _L, _L)]
                xv[r, pl.ds(i * _L, _L)] = (
                    (y - mean_v) * scale * wv[pl.ds(i * _L, _L)]
                    + bv[pl.ds(i * _L, _L)])
            return 0

        lax.fori_loop(0, chunk_rows, row_body, 0)
        base = out0 + c * chunk_rows
        pltpu.async_copy(xv, out_hbm.at[pl.ds(base, chunk_rows)], ssem)

    fetch(0, 0)
    fetch(1, 1)

    def pair_body(p, _):
        c = p * 2
        for b in range(2):
            compute_store(c + b, b)

            @pl.when(c + b + 2 < n_chunks)
            def _():
                drain_store(b)
                fetch(c + b + 2, b)

        return 0

    lax.fori_loop(0, n_chunks // 2, pair_body, 0)
    drain_store(0)
    drain_store(1)


def _sc_rows(x2, enc, ln_w, ln_b, row_start, n_sc_rows, seq, n_attr,
             chunk_rows=16):
    info = plsc.get_sparse_core_info()
    n_workers = info.num_cores * info.num_subcores
    rows_per_worker = n_sc_rows // n_workers

    mesh = plsc.VectorSubcoreMesh(core_axis_name="c", subcore_axis_name="s")
    body = functools.partial(
        _sc_body, row_start, rows_per_worker, chunk_rows, n_attr, seq)
    return pl.kernel(
        body,
        mesh=mesh,
        out_type=jax.ShapeDtypeStruct((n_sc_rows, n_attr), jnp.float32),
        compiler_params=pltpu.CompilerParams(needs_layout_passes=False),
        scratch_types=[
            pltpu.VMEM((chunk_rows, n_attr), jnp.float32),
            pltpu.VMEM((chunk_rows, n_attr), jnp.float32),
            pltpu.VMEM((chunk_rows, n_attr), jnp.float32),
            pltpu.VMEM((chunk_rows, n_attr), jnp.float32),
            pltpu.VMEM((n_attr,), jnp.float32),
            pltpu.VMEM((n_attr,), jnp.float32),
            pltpu.SemaphoreType.DMA,
            pltpu.SemaphoreType.DMA,
            pltpu.SemaphoreType.DMA,
            pltpu.SemaphoreType.DMA,
        ],
    )(x2, enc, ln_w, ln_b)


@jax.jit
def _tempo_enc_sc_full(x, enc_table, ln_w, ln_b):
    batch, seq, n_attr = x.shape
    n_rows = batch * seq
    x2 = x.reshape(n_rows, n_attr)
    enc = enc_table[:seq]
    out = _sc_rows(x2, enc, ln_w, ln_b, 0, n_rows, seq, n_attr)
    return out.reshape(batch, seq, n_attr)


def _ln_body_ab(x_ref, enc_ref, w_ref, b_ref, o_ref):
    y = x_ref[...] + enc_ref[...]
    mean = jnp.mean(y, axis=-1, keepdims=True)
    yc = y - mean
    var = jnp.mean(yc * yc, axis=-1, keepdims=True)
    o_ref[...] = yc * jax.lax.rsqrt(var + _EPS) * w_ref[...] + b_ref[...]


@functools.partial(jax.jit, static_argnames=("bs",))
def _tempo_enc_tc_allbatch(x, enc_table, ln_w, ln_b, bs=512):
    batch, seq, n_attr = x.shape
    enc = enc_table[:seq].reshape(1, seq, n_attr)
    w2 = ln_w.reshape(1, 1, n_attr)
    b2 = ln_b.reshape(1, 1, n_attr)
    return pl.pallas_call(
        _ln_body_ab,
        grid=(seq // bs,),
        in_specs=[
            pl.BlockSpec((batch, bs, n_attr), lambda s: (0, s, 0)),
            pl.BlockSpec((1, bs, n_attr), lambda s: (0, s, 0)),
            pl.BlockSpec((1, 1, n_attr), lambda s: (0, 0, 0)),
            pl.BlockSpec((1, 1, n_attr), lambda s: (0, 0, 0)),
        ],
        out_specs=pl.BlockSpec((batch, bs, n_attr), lambda s: (0, s, 0)),
        out_shape=jax.ShapeDtypeStruct(x.shape, x.dtype),
        compiler_params=pltpu.CompilerParams(
            dimension_semantics=("arbitrary",),
        ),
    )(x, enc, w2, b2)


def kernel(x, enc_table, ln_w, ln_b):
    return _tempo_enc_tc_allbatch(x, enc_table, ln_w, ln_b, bs=512)
